# Initial kernel scaffold; baseline (speedup 1.0000x reference)
import jax
import jax.numpy as jnp
from jax.experimental import pallas as pl


def kernel(z, W):
    raise NotImplementedError("write your pallas kernel here")


# speedup vs baseline: 1.3083x; 1.3083x over previous
import functools

import jax
import jax.numpy as jnp
from jax import lax
from jax.experimental import pallas as pl
from jax.experimental.pallas import tpu as pltpu
from jax.experimental.pallas import tpu_sc as plsc

N_CB = 8
K = 1024
CD = 128
NTOK = 16384
TB = 512


def _dist_argmin_body(z_ref, w_ref, idx_ref, loss_ref):
    i = pl.program_id(0)
    t = pl.program_id(1)

    @pl.when((i == 0) & (t == 0))
    def _():
        loss_ref[...] = jnp.zeros_like(loss_ref)

    r = z_ref[0]
    w = w_ref[0]
    r2 = jnp.sum(r * r, axis=1, keepdims=True)
    w2 = jnp.sum(w * w, axis=1)
    lhs = (2.0 * r).astype(jnp.bfloat16)
    rhs = w.astype(jnp.bfloat16)
    mm = lax.dot_general(lhs, rhs, (((1,), (1,)), ((), ())),
                         preferred_element_type=jnp.float32)
    dist = (r2 + w2[None, :]) - mm
    mind = jnp.min(dist, axis=1, keepdims=True)
    ids = lax.broadcasted_iota(jnp.int32, dist.shape, 1)
    idx_ref[0, 0, :] = jnp.min(
        jnp.where(dist == mind, ids, jnp.int32(2**30)), axis=1)
    loss_ref[...] = loss_ref[...] + jnp.sum(mind)


def _dist_argmin(zt, W):
    return pl.pallas_call(
        _dist_argmin_body,
        grid=(N_CB, NTOK // TB),
        in_specs=[
            pl.BlockSpec((1, TB, CD), lambda i, t: (i, t, 0)),
            pl.BlockSpec((1, K, CD), lambda i, t: (i, 0, 0)),
        ],
        out_specs=[
            pl.BlockSpec((1, 1, TB), lambda i, t: (i, 0, t)),
            pl.BlockSpec((1, 1), lambda i, t: (0, 0)),
        ],
        out_shape=[
            jax.ShapeDtypeStruct((N_CB, 1, NTOK), jnp.int32),
            jax.ShapeDtypeStruct((1, 1), jnp.float32),
        ],
    )(zt, W)


_SC_INFO = plsc.get_sparse_core_info()
_NC = _SC_INFO.num_cores
_NS = _SC_INFO.num_subcores
_NW = _NC * _NS
_ROWS = N_CB * NTOK
_PER_W = _ROWS // _NW
_CHUNK = 128
_NITER = _PER_W // _CHUNK


@functools.partial(
    pl.kernel,
    mesh=plsc.VectorSubcoreMesh(core_axis_name="c", subcore_axis_name="s"),
    out_type=jax.ShapeDtypeStruct((_ROWS, CD), jnp.float32),
    scratch_types=[
        pltpu.VMEM((_CHUNK,), jnp.int32),
        pltpu.VMEM((_CHUNK, CD), jnp.float32),
        pltpu.SemaphoreType.DMA,
    ],
)
def _sc_gather(table_hbm, idx_hbm, out_hbm, idx_v, rows_v, sem):
    wid = lax.axis_index("s") * _NC + lax.axis_index("c")
    base = wid * _PER_W

    def step(j, carry):
        off = base + j * _CHUNK
        pltpu.sync_copy(idx_hbm.at[pl.ds(off, _CHUNK)], idx_v)
        pltpu.async_copy(table_hbm.at[idx_v], rows_v, sem).wait()
        pltpu.sync_copy(rows_v, out_hbm.at[pl.ds(off, _CHUNK)])
        return carry

    lax.fori_loop(0, _NITER, step, 0)


def kernel(z, W):
    Bz, Tz, Dz = z.shape
    zt = z.reshape(NTOK, N_CB, CD).transpose(1, 0, 2)
    idx8, loss = _dist_argmin(zt, W)
    idx2 = idx8.reshape(N_CB, NTOK)

    gidx = (idx2 + (jnp.arange(N_CB, dtype=jnp.int32) * K)[:, None])
    gidx_flat = gidx.T.reshape(_ROWS)

    zq_rows = _sc_gather(W.reshape(N_CB * K, CD), gidx_flat)
    z_q = zq_rows.reshape(Bz, Tz, Dz)

    indices = idx2.reshape(N_CB, Bz, Tz).transpose(1, 0, 2)
    total_loss = loss[0, 0] / jnp.float32(NTOK * CD)
    return z_q, indices, total_loss
